# Initial kernel scaffold; baseline (speedup 1.0000x reference)
import jax
import jax.numpy as jnp
from jax.experimental import pallas as pl


def kernel(hidden_states, router_weight, e_score_correction_bias, w_gate, w_up, w_down):
    raise NotImplementedError("write your pallas kernel here")


# speedup vs baseline: 3.1040x; 3.1040x over previous
import functools

import jax
import jax.numpy as jnp
from jax.experimental import pallas as pl
from jax.experimental.pallas import tpu as pltpu

B = 1
S = 2048
HIDDEN = 2048
FFN = 512
N_ROUTED = 64
N_EXP = 80
EPAD = 128
TOPK = 2
SCALE = 1.0
EPS = 1e-20

TM = 64
M_PAD = 8192
N_TILES = M_PAD // TM
S_TILE = 256


def _router_body(x_ref, rwt_ref, bias_ref, out_ref):
    logits = jax.lax.dot_general(
        x_ref[...], rwt_ref[...], (((1,), (0,)), ((), ())),
        preferred_element_type=jnp.float32)
    scores = jax.nn.sigmoid(logits)
    c = scores + bias_ref[0:1, :]
    iota = jax.lax.broadcasted_iota(jnp.int32, (S_TILE, EPAD), 1)
    m0 = jnp.max(c, axis=1, keepdims=True)
    a0 = jnp.min(jnp.where(c == m0, iota, EPAD), axis=1, keepdims=True)
    c1 = jnp.where(iota == a0, -1e30, c)
    m1 = jnp.max(c1, axis=1, keepdims=True)
    a1 = jnp.min(jnp.where(c1 == m1, iota, EPAD), axis=1, keepdims=True)
    s0 = jnp.sum(jnp.where(iota == a0, scores, 0.0), axis=1, keepdims=True)
    s1 = jnp.sum(jnp.where(iota == a1, scores, 0.0), axis=1, keepdims=True)
    den = s0 + s1 + EPS
    w0 = s0 / den * SCALE
    w1 = s1 / den * SCALE
    z0 = a0 >= N_ROUTED
    z1 = a1 >= N_ROUTED
    rw0 = jnp.where(z0, 0.0, w0)
    rw1 = jnp.where(z1, 0.0, w1)
    zw = jnp.where(z0, w0, 0.0) + jnp.where(z1, w1, 0.0)
    e0 = jnp.where(z0, N_ROUTED, a0).astype(jnp.float32)
    e1 = jnp.where(z1, N_ROUTED, a1).astype(jnp.float32)
    li = jax.lax.broadcasted_iota(jnp.int32, (S_TILE, EPAD), 1)
    out = jnp.where(li == 0, e0, 0.0)
    out = jnp.where(li == 1, e1, out)
    out = jnp.where(li == 2, rw0, out)
    out = jnp.where(li == 3, rw1, out)
    out = jnp.where(li == 4, zw, out)
    out_ref[...] = out


def _router(x, rwt_pad, bias_pad):
    return pl.pallas_call(
        _router_body,
        grid=(S // S_TILE,),
        in_specs=[
            pl.BlockSpec((S_TILE, HIDDEN), lambda i: (i, 0)),
            pl.BlockSpec((HIDDEN, EPAD), lambda i: (0, 0)),
            pl.BlockSpec((8, EPAD), lambda i: (0, 0)),
        ],
        out_specs=pl.BlockSpec((S_TILE, EPAD), lambda i: (i, 0)),
        out_shape=jax.ShapeDtypeStruct((S, EPAD), jnp.float32),
    )(x, rwt_pad, bias_pad)


def _gmm_body(te_ref, xs_ref, wg_ref, wu_ref, wd_ref, ys_ref):
    t = pl.program_id(0)
    e = te_ref[t]

    @pl.when(e >= 0)
    def _():
        xt = xs_ref[...]
        g = jax.lax.dot_general(xt, wg_ref[0], (((1,), (1,)), ((), ())),
                                preferred_element_type=jnp.float32)
        u = jax.lax.dot_general(xt, wu_ref[0], (((1,), (1,)), ((), ())),
                                preferred_element_type=jnp.float32)
        h = g * jax.nn.sigmoid(g) * u
        y = jax.lax.dot_general(h, wd_ref[0], (((1,), (1,)), ((), ())),
                                preferred_element_type=jnp.float32)
        ys_ref[...] = y

    @pl.when(jnp.logical_and(e < 0, t == 0))
    def _():
        ys_ref[...] = jnp.zeros_like(ys_ref)


def _gmm(te, xs, w_gate, w_up, w_down):
    def clamp(e):
        return jnp.clip(e, 0, N_ROUTED - 1)

    grid_spec = pltpu.PrefetchScalarGridSpec(
        num_scalar_prefetch=1,
        grid=(N_TILES,),
        in_specs=[
            pl.BlockSpec((TM, HIDDEN), lambda t, te: (t, 0)),
            pl.BlockSpec((1, FFN, HIDDEN), lambda t, te: (clamp(te[t]), 0, 0)),
            pl.BlockSpec((1, FFN, HIDDEN), lambda t, te: (clamp(te[t]), 0, 0)),
            pl.BlockSpec((1, HIDDEN, FFN), lambda t, te: (clamp(te[t]), 0, 0)),
        ],
        out_specs=pl.BlockSpec((TM, HIDDEN), lambda t, te: (t, 0)),
    )
    return pl.pallas_call(
        _gmm_body,
        grid_spec=grid_spec,
        out_shape=jax.ShapeDtypeStruct((M_PAD, HIDDEN), jnp.float32),
    )(te, xs, w_gate, w_up, w_down)


def _dispatch(e0, e1):
    keys = jnp.concatenate([e0, e1])
    order = jnp.argsort(keys)
    ks = keys[order]
    first = jnp.searchsorted(ks, ks, side="left")
    rank_sorted = jnp.arange(2 * S, dtype=jnp.int32) - first.astype(jnp.int32)
    rank = jnp.zeros(2 * S, jnp.int32).at[order].set(rank_sorted)
    counts = jnp.bincount(keys, length=N_ROUTED + 1).astype(jnp.int32)
    rc = ((counts[:N_ROUTED] + TM - 1) // TM) * TM
    offsets = jnp.concatenate([jnp.zeros(1, jnp.int32), jnp.cumsum(rc)])
    valid = keys < N_ROUTED
    pos = jnp.where(valid, offsets[jnp.minimum(keys, N_ROUTED - 1)] + rank, 0)
    token_of_row = jnp.zeros(M_PAD, jnp.int32).at[
        jnp.where(valid, pos, M_PAD - 1)].set(
        jnp.where(valid, jnp.arange(2 * S, dtype=jnp.int32) % S, 0))
    tile_start = offsets // TM
    t_iota = jnp.arange(N_TILES)
    te = jnp.sum(tile_start[None, 1:N_ROUTED] <= t_iota[:, None], axis=1)
    te = jnp.where(t_iota < tile_start[N_ROUTED], te, -1).astype(jnp.int32)
    return pos, token_of_row, te


def kernel(hidden_states, router_weight, e_score_correction_bias, w_gate, w_up, w_down):
    x = hidden_states.reshape(-1, HIDDEN).astype(jnp.float32)
    rwt_pad = jnp.zeros((HIDDEN, EPAD), jnp.float32).at[:, :N_EXP].set(router_weight.T)
    bias_pad = jnp.full((8, EPAD), -1e30, jnp.float32).at[:, :N_EXP].set(
        e_score_correction_bias[None, :])
    r = _router(x, rwt_pad, bias_pad)
    e0 = r[:, 0].astype(jnp.int32)
    e1 = r[:, 1].astype(jnp.int32)
    rw0 = r[:, 2]
    rw1 = r[:, 3]
    zw = r[:, 4]

    pos, token_of_row, te = _dispatch(e0, e1)
    xs = jnp.take(x, token_of_row, axis=0)
    ys = _gmm(te, xs, w_gate, w_up, w_down)
    p0, p1 = pos[:S], pos[S:]
    out = zw[:, None] * x + rw0[:, None] * jnp.take(ys, p0, axis=0) \
        + rw1[:, None] * jnp.take(ys, p1, axis=0)
    return out.reshape(B, S, HIDDEN)
